# Initial kernel scaffold; baseline (speedup 1.0000x reference)
import functools

import jax
import jax.numpy as jnp
from jax import lax
from jax.experimental import pallas as pl
from jax.experimental.pallas import tpu as pltpu
from jax.experimental.pallas import tpu_sc as plsc

N = 10000
E = 320000
D = 128
R = 8

NC = 2
NS = 16
LANES = 16
HALF = D // NC

CHUNK = 128
SUB = 16
EPT = 20480
E_PAD = EPT * NS
NSUPER = EPT // (CHUNK * SUB)
N_CHUNK_ROWS = E_PAD // CHUNK
DUMP = 480
ACC_ROWS = 2 * N + DUMP
ROWS_PER_TILE = ACC_ROWS // NS


def _xtab_body(x_ref, r_ref, o_ref):
    o_ref[...] = (x_ref[...] * r_ref[...])[None]


def _build_xtab(n_in_feats, r_all):
    xbn = 2500
    return pl.pallas_call(
        _xtab_body,
        grid=(R + 1, N // xbn),
        in_specs=[
            pl.BlockSpec((xbn, D), lambda r, i: (i, 0)),
            pl.BlockSpec((1, D), lambda r, i: (r, 0)),
        ],
        out_specs=pl.BlockSpec((1, xbn, D), lambda r, i: (r, i, 0)),
        out_shape=jax.ShapeDtypeStruct((R + 1, N, D), jnp.float32),
    )(n_in_feats, r_all)


def _sc_body(x_hbm, gidx_hbm, sidx_hbm, norm_hbm, zeros_hbm, out_hbm,
             gidxb, sidxb, normb, gidx_buf, rows, acc, gsem):
    c = lax.axis_index("c")
    s = lax.axis_index("s")

    pltpu.sync_copy(zeros_hbm, rows)
    zrow0 = s * ROWS_PER_TILE
    for z in range(ROWS_PER_TILE // CHUNK):
        pltpu.sync_copy(rows, acc.at[pl.ds(zrow0 + z * CHUNK, CHUNK), :])
    plsc.subcore_barrier()

    iota = lax.iota(jnp.int32, LANES)
    cols = [iota + v * LANES for v in range(HALF // LANES)]

    def super_body(sup, carry):
        crow0 = s * (NSUPER * SUB) + sup * SUB
        pltpu.sync_copy(gidx_hbm.at[pl.ds(crow0, SUB), :], gidxb)
        pltpu.sync_copy(sidx_hbm.at[pl.ds(crow0, SUB), :], sidxb)
        pltpu.sync_copy(norm_hbm.at[pl.ds(crow0, SUB), :], normb)
        for j in range(SUB):
            def gi_body(b, carry2):
                base = b * LANES
                gidx_buf[pl.ds(base, LANES)] = (
                    gidxb[j, pl.ds(base, LANES)] + c)
                return carry2
            lax.fori_loop(0, CHUNK // LANES, gi_body, 0, unroll=True)
            pltpu.async_copy(x_hbm.at[gidx_buf], rows, gsem).wait()

            jvec = jnp.full((LANES,), j, dtype=jnp.int32)

            def e_body(e, carry2):
                evec = jnp.full((LANES,), e, dtype=jnp.int32)
                nrm = plsc.load_gather(normb, [jvec, evec])
                for colv in cols:
                    x = plsc.load_gather(rows, [evec, colv])
                    plsc.store_scatter(rows, [evec, colv], x * nrm)
                return carry2
            lax.fori_loop(0, CHUNK, e_body, 0)

            pltpu.sync_copy(rows, acc.at[sidxb.at[j]], add=True)
        return carry

    lax.fori_loop(0, NSUPER, super_body, 0)
    plsc.subcore_barrier()

    pltpu.sync_copy(acc.at[pl.ds(zrow0, ROWS_PER_TILE), :],
                    out_hbm.at[c, pl.ds(zrow0, ROWS_PER_TILE), :])


@functools.partial(
    pl.kernel,
    out_type=jax.ShapeDtypeStruct((NC, ACC_ROWS, HALF), jnp.float32),
    mesh=plsc.VectorSubcoreMesh(core_axis_name="c", subcore_axis_name="s"),
    scratch_types=[
        pltpu.VMEM((SUB, CHUNK), jnp.int32),
        pltpu.VMEM((SUB, CHUNK), jnp.int32),
        pltpu.VMEM((SUB, CHUNK), jnp.float32),
        pltpu.VMEM((CHUNK,), jnp.int32),
        pltpu.VMEM((CHUNK, HALF), jnp.float32),
        pltpu.VMEM_SHARED((ACC_ROWS, HALF), jnp.float32),
        pltpu.SemaphoreType.DMA,
    ],
)
def _sc_edge_kernel(x_hbm, gidx_hbm, sidx_hbm, norm_hbm, zeros_hbm, out_hbm,
                    gidxb, sidxb, normb, gidx_buf, rows, acc, gsem):
    _sc_body(x_hbm, gidx_hbm, sidx_hbm, norm_hbm, zeros_hbm, out_hbm,
             gidxb, sidxb, normb, gidx_buf, rows, acc, gsem)


def _final_body(x_ref, a00, a01, a10, a11, wt_ref, wst_ref, loop_ref,
                rf_ref, wrt_ref, o_ref, r_ref):
    i = pl.program_id(0)
    xs = x_ref[...] * loop_ref[...]
    acc = jnp.dot(xs, wst_ref[...], preferred_element_type=jnp.float32)
    acc += jnp.dot(a00[0], wt_ref[0, 0], preferred_element_type=jnp.float32)
    acc += jnp.dot(a01[0], wt_ref[0, 1], preferred_element_type=jnp.float32)
    acc += jnp.dot(a10[0], wt_ref[1, 0], preferred_element_type=jnp.float32)
    acc += jnp.dot(a11[0], wt_ref[1, 1], preferred_element_type=jnp.float32)
    o_ref[...] = jnp.tanh(acc * (1.0 / 3.0))

    @pl.when(i == 0)
    def _():
        r_ref[...] = jnp.dot(rf_ref[...], wrt_ref[...],
                             preferred_element_type=jnp.float32)


def _final(n_in_feats, out_sc, wt, ws_t, loop_rel, r_feats, wr_t):
    bn = 1000
    nb = N // bn

    def a_spec(cc, mm):
        return pl.BlockSpec((1, bn, HALF),
                            lambda i, cc=cc, mm=mm: (cc, mm * nb + i, 0))

    return pl.pallas_call(
        _final_body,
        grid=(nb,),
        in_specs=[
            pl.BlockSpec((bn, D), lambda i: (i, 0)),
            a_spec(0, 0), a_spec(0, 1), a_spec(1, 0), a_spec(1, 1),
            pl.BlockSpec((NC, 2, HALF, D), lambda i: (0, 0, 0, 0)),
            pl.BlockSpec((D, D), lambda i: (0, 0)),
            pl.BlockSpec((1, D), lambda i: (0, 0)),
            pl.BlockSpec((R, D), lambda i: (0, 0)),
            pl.BlockSpec((D, D), lambda i: (0, 0)),
        ],
        out_specs=[
            pl.BlockSpec((bn, D), lambda i: (i, 0)),
            pl.BlockSpec((R, D), lambda i: (0, 0)),
        ],
        out_shape=[
            jax.ShapeDtypeStruct((N, D), jnp.float32),
            jax.ShapeDtypeStruct((R, D), jnp.float32),
        ],
    )(n_in_feats, out_sc, out_sc, out_sc, out_sc, wt, ws_t, loop_rel,
      r_feats, wr_t)


def kernel(n_in_feats, r_feats, edge_index, etype, norm, out_edges_mask,
           W_O, W_I, W_S, W_R, loop_rel):
    r_all = jnp.concatenate([r_feats, loop_rel], axis=0)
    xtab = _build_xtab(n_in_feats, r_all)
    xflat = xtab.reshape((R + 1) * N * NC, HALF)

    src = edge_index[0]
    dst = edge_index[1]
    gidx = (etype * N + src) * NC
    sidx = out_edges_mask * N + dst
    norm_e = norm[:, 0]

    pad = E_PAD - E
    pad_i = jnp.arange(pad, dtype=jnp.int32)
    gidx_p = jnp.concatenate([gidx, (pad_i % N) * NC])
    sidx_p = jnp.concatenate([sidx, 2 * N + (pad_i % DUMP)])
    norm_p = jnp.concatenate([norm_e, jnp.zeros((pad,), jnp.float32)])

    out_sc = _sc_edge_kernel(
        xflat,
        gidx_p.reshape(N_CHUNK_ROWS, CHUNK),
        sidx_p.reshape(N_CHUNK_ROWS, CHUNK),
        norm_p.reshape(N_CHUNK_ROWS, CHUNK),
        jnp.zeros((CHUNK, HALF), jnp.float32),
    )

    wt = jnp.stack([
        jnp.stack([W_I[:, :HALF].T, W_O[:, :HALF].T]),
        jnp.stack([W_I[:, HALF:].T, W_O[:, HALF:].T]),
    ])

    n_out, r_out = _final(n_in_feats, out_sc, wt, W_S.T, loop_rel,
                          r_feats, W_R.T)
    return n_out, r_out


# speedup vs baseline: 1.6407x; 1.6407x over previous
import functools

import jax
import jax.numpy as jnp
from jax import lax
from jax.experimental import pallas as pl
from jax.experimental.pallas import tpu as pltpu
from jax.experimental.pallas import tpu_sc as plsc

N = 10000
E = 320000
D = 128
R = 8

NC = 2
NS = 16
NW = NC * NS
LANES = 16

CPT = D // NW
ROWS = 2 * N
CH = 512
NCHUNK = E // CH
RP = 16


def _sc_body(nin_hbm, rall_hbm, ps_hbm, sidx_hbm, norm_hbm, out_hbm,
             nin, rall, psb, sidxb, normb, acc):
    c = lax.axis_index("c")
    s = lax.axis_index("s")
    tid = c * NS + s

    pltpu.sync_copy(nin_hbm.at[pl.ds(tid * CPT * N, CPT * N)], nin)
    pltpu.sync_copy(rall_hbm.at[pl.ds(tid * CPT * RP, CPT * RP)], rall)

    zero = jnp.zeros((LANES,), jnp.float32)

    def z_body(i, carry):
        acc[pl.ds(i * LANES, LANES)] = zero
        return carry
    lax.fori_loop(0, CPT * ROWS // LANES, z_body, 0)

    def c_body(ch, carry):
        base = ch * CH
        pltpu.sync_copy(ps_hbm.at[pl.ds(base, CH)], psb)
        pltpu.sync_copy(sidx_hbm.at[pl.ds(base, CH)], sidxb)
        pltpu.sync_copy(norm_hbm.at[pl.ds(base, CH)], normb)

        def g_body(g, carry2):
            sl = pl.ds(g * LANES, LANES)
            ps_v = psb[sl]
            src_v = ps_v & 16383
            ret_v = (ps_v >> 14) & 15
            sidx_v = sidxb[sl]
            norm_v = normb[sl]
            for col in range(CPT):
                nin_v = plsc.load_gather(nin, [src_v + col * N])
                ral_v = plsc.load_gather(rall, [ret_v + col * RP])
                val = nin_v * ral_v * norm_v
                plsc.addupdate_scatter(acc, [sidx_v + col * ROWS], val)
            return carry2
        lax.fori_loop(0, CH // LANES, g_body, 0)
        return carry
    lax.fori_loop(0, NCHUNK, c_body, 0)

    pltpu.sync_copy(acc, out_hbm.at[pl.ds(tid * CPT * ROWS, CPT * ROWS)])


@functools.partial(
    pl.kernel,
    out_type=jax.ShapeDtypeStruct((D * ROWS,), jnp.float32),
    mesh=plsc.VectorSubcoreMesh(core_axis_name="c", subcore_axis_name="s"),
    compiler_params=pltpu.CompilerParams(needs_layout_passes=False),
    scratch_types=[
        pltpu.VMEM((CPT * N,), jnp.float32),
        pltpu.VMEM((CPT * RP,), jnp.float32),
        pltpu.VMEM((CH,), jnp.int32),
        pltpu.VMEM((CH,), jnp.int32),
        pltpu.VMEM((CH,), jnp.float32),
        pltpu.VMEM((CPT * ROWS,), jnp.float32),
    ],
)
def _sc_edge_kernel(nin_hbm, rall_hbm, ps_hbm, sidx_hbm, norm_hbm, out_hbm,
                    nin, rall, psb, sidxb, normb, acc):
    _sc_body(nin_hbm, rall_hbm, ps_hbm, sidx_hbm, norm_hbm, out_hbm,
             nin, rall, psb, sidxb, normb, acc)


def _final_body(x_ref, ai_ref, ao_ref, wit_ref, wot_ref, wst_ref, loop_ref,
                rf_ref, wrt_ref, o_ref, r_ref):
    i = pl.program_id(0)
    xs = x_ref[...] * loop_ref[...]
    acc = jnp.dot(xs, wst_ref[...], preferred_element_type=jnp.float32)
    acc += jnp.dot(ai_ref[...], wit_ref[...], preferred_element_type=jnp.float32)
    acc += jnp.dot(ao_ref[...], wot_ref[...], preferred_element_type=jnp.float32)
    o_ref[...] = jnp.tanh(acc * (1.0 / 3.0))

    @pl.when(i == 0)
    def _():
        r_ref[...] = jnp.dot(rf_ref[...], wrt_ref[...],
                             preferred_element_type=jnp.float32)


def _final(n_in_feats, a_t, wi_t, wo_t, ws_t, loop_rel, r_feats, wr_t):
    bn = 1000
    nb = N // bn
    return pl.pallas_call(
        _final_body,
        grid=(nb,),
        in_specs=[
            pl.BlockSpec((bn, D), lambda i: (i, 0)),
            pl.BlockSpec((bn, D), lambda i: (i, 0)),
            pl.BlockSpec((bn, D), lambda i: (nb + i, 0)),
            pl.BlockSpec((D, D), lambda i: (0, 0)),
            pl.BlockSpec((D, D), lambda i: (0, 0)),
            pl.BlockSpec((D, D), lambda i: (0, 0)),
            pl.BlockSpec((1, D), lambda i: (0, 0)),
            pl.BlockSpec((R, D), lambda i: (0, 0)),
            pl.BlockSpec((D, D), lambda i: (0, 0)),
        ],
        out_specs=[
            pl.BlockSpec((bn, D), lambda i: (i, 0)),
            pl.BlockSpec((R, D), lambda i: (0, 0)),
        ],
        out_shape=[
            jax.ShapeDtypeStruct((N, D), jnp.float32),
            jax.ShapeDtypeStruct((R, D), jnp.float32),
        ],
    )(n_in_feats, a_t, a_t, wi_t, wo_t, ws_t, loop_rel, r_feats, wr_t)


def kernel(n_in_feats, r_feats, edge_index, etype, norm, out_edges_mask,
           W_O, W_I, W_S, W_R, loop_rel):
    r_all = jnp.concatenate([r_feats, loop_rel], axis=0)
    nin_t = n_in_feats.T.reshape(-1)
    rall_t = jnp.pad(r_all.T, ((0, 0), (0, RP - (R + 1)))).reshape(-1)

    src = edge_index[0]
    dst = edge_index[1]
    ps = src | (etype << 14)
    sidx = out_edges_mask * N + dst
    norm_e = norm[:, 0]

    flat = _sc_edge_kernel(nin_t, rall_t, ps, sidx, norm_e)
    a_t = flat.reshape(D, ROWS).T

    n_out, r_out = _final(n_in_feats, a_t, W_I.T, W_O.T, W_S.T, loop_rel,
                          r_feats, W_R.T)
    return n_out, r_out


# speedup vs baseline: 2.8613x; 1.7439x over previous
import functools

import jax
import jax.numpy as jnp
from jax import lax
from jax.experimental import pallas as pl
from jax.experimental.pallas import tpu as pltpu
from jax.experimental.pallas import tpu_sc as plsc

N = 10000
E = 320000
D = 128
R = 8

NC = 2
NS = 16
NW = NC * NS
LANES = 16

CPT = D // NW
ROWS = 2 * N
CH = 2560
NCHUNK = E // CH
RP = 16


def _sc_body(nin_hbm, rall_hbm, meta_hbm, norm_hbm, out_hbm,
             nin, rall, metab, normb, acc):
    c = lax.axis_index("c")
    s = lax.axis_index("s")
    tid = c * NS + s

    pltpu.sync_copy(nin_hbm.at[pl.ds(tid * CPT * N, CPT * N)], nin)
    pltpu.sync_copy(rall_hbm.at[pl.ds(tid * CPT * RP, CPT * RP)], rall)

    zero = jnp.zeros((LANES,), jnp.float32)

    def z_body(i, carry):
        acc[pl.ds(i * LANES, LANES)] = zero
        return carry
    lax.fori_loop(0, CPT * ROWS // LANES, z_body, 0)

    def c_body(ch, carry):
        pltpu.sync_copy(meta_hbm.at[pl.ds(ch * 2 * CH, 2 * CH)], metab)
        pltpu.sync_copy(norm_hbm.at[pl.ds(ch * CH, CH)], normb)

        def g_body(g, carry2):
            sl = pl.ds(g * LANES, LANES)
            ps_v = metab[sl]
            src_v = ps_v & 16383
            ret_v = (ps_v >> 14) & 15
            sidx_v = metab[pl.ds(CH + g * LANES, LANES)]
            norm_v = normb[sl]
            for col in range(CPT):
                nin_v = plsc.load_gather(nin, [src_v + col * N])
                ral_v = plsc.load_gather(rall, [ret_v + col * RP])
                val = nin_v * ral_v * norm_v
                plsc.addupdate_scatter(acc, [sidx_v + col * ROWS], val)
            return carry2
        lax.fori_loop(0, CH // LANES, g_body, 0)
        return carry
    lax.fori_loop(0, NCHUNK, c_body, 0)

    pltpu.sync_copy(acc, out_hbm.at[pl.ds(tid * CPT * ROWS, CPT * ROWS)])


@functools.partial(
    pl.kernel,
    out_type=jax.ShapeDtypeStruct((D * ROWS,), jnp.float32),
    mesh=plsc.VectorSubcoreMesh(core_axis_name="c", subcore_axis_name="s"),
    compiler_params=pltpu.CompilerParams(needs_layout_passes=False),
    scratch_types=[
        pltpu.VMEM((CPT * N,), jnp.float32),
        pltpu.VMEM((CPT * RP,), jnp.float32),
        pltpu.VMEM((2 * CH,), jnp.int32),
        pltpu.VMEM((CH,), jnp.float32),
        pltpu.VMEM((CPT * ROWS,), jnp.float32),
    ],
)
def _sc_edge_kernel(nin_hbm, rall_hbm, meta_hbm, norm_hbm, out_hbm,
                    nin, rall, metab, normb, acc):
    _sc_body(nin_hbm, rall_hbm, meta_hbm, norm_hbm, out_hbm,
             nin, rall, metab, normb, acc)


def _final_body(x_ref, ai_ref, ao_ref, wit_ref, wot_ref, wst_ref, loop_ref,
                rf_ref, wrt_ref, o_ref, r_ref):
    i = pl.program_id(0)
    xs = x_ref[...] * loop_ref[...]
    acc = jnp.dot(xs, wst_ref[...], preferred_element_type=jnp.float32)
    acc += jnp.dot(ai_ref[...], wit_ref[...], preferred_element_type=jnp.float32)
    acc += jnp.dot(ao_ref[...], wot_ref[...], preferred_element_type=jnp.float32)
    o_ref[...] = jnp.tanh(acc * (1.0 / 3.0))

    @pl.when(i == 0)
    def _():
        r_ref[...] = jnp.dot(rf_ref[...], wrt_ref[...],
                             preferred_element_type=jnp.float32)


def _final(n_in_feats, a_t, wi_t, wo_t, ws_t, loop_rel, r_feats, wr_t):
    bn = 1000
    nb = N // bn
    return pl.pallas_call(
        _final_body,
        grid=(nb,),
        in_specs=[
            pl.BlockSpec((bn, D), lambda i: (i, 0)),
            pl.BlockSpec((bn, D), lambda i: (i, 0)),
            pl.BlockSpec((bn, D), lambda i: (nb + i, 0)),
            pl.BlockSpec((D, D), lambda i: (0, 0)),
            pl.BlockSpec((D, D), lambda i: (0, 0)),
            pl.BlockSpec((D, D), lambda i: (0, 0)),
            pl.BlockSpec((1, D), lambda i: (0, 0)),
            pl.BlockSpec((R, D), lambda i: (0, 0)),
            pl.BlockSpec((D, D), lambda i: (0, 0)),
        ],
        out_specs=[
            pl.BlockSpec((bn, D), lambda i: (i, 0)),
            pl.BlockSpec((R, D), lambda i: (0, 0)),
        ],
        out_shape=[
            jax.ShapeDtypeStruct((N, D), jnp.float32),
            jax.ShapeDtypeStruct((R, D), jnp.float32),
        ],
    )(n_in_feats, a_t, a_t, wi_t, wo_t, ws_t, loop_rel, r_feats, wr_t)


def kernel(n_in_feats, r_feats, edge_index, etype, norm, out_edges_mask,
           W_O, W_I, W_S, W_R, loop_rel):
    r_all = jnp.concatenate([r_feats, loop_rel], axis=0)
    nin_t = n_in_feats.T.reshape(-1)
    rall_t = jnp.pad(r_all.T, ((0, 0), (0, RP - (R + 1)))).reshape(-1)

    src = edge_index[0]
    dst = edge_index[1]
    ps = src | (etype << 14)
    sidx = out_edges_mask * N + dst
    meta = jnp.stack([ps.reshape(NCHUNK, CH), sidx.reshape(NCHUNK, CH)],
                     axis=1).reshape(-1)
    norm_e = norm[:, 0]

    flat = _sc_edge_kernel(nin_t, rall_t, meta, norm_e)
    a_t = flat.reshape(D, ROWS).T

    n_out, r_out = _final(n_in_feats, a_t, W_I.T, W_O.T, W_S.T, loop_rel,
                          r_feats, W_R.T)
    return n_out, r_out


# speedup vs baseline: 2.9362x; 1.0262x over previous
import functools

import jax
import jax.numpy as jnp
from jax import lax
from jax.experimental import pallas as pl
from jax.experimental.pallas import tpu as pltpu
from jax.experimental.pallas import tpu_sc as plsc

N = 10000
E = 320000
D = 128
R = 8

NC = 2
NS = 16
NW = NC * NS
LANES = 16

CPT = D // NW
ROWS = 2 * N
CH = 3200
NCHUNK = E // CH
RP = 16


def _sc_body(nin_hbm, rall_hbm, meta_hbm, norm_hbm, out_hbm,
             nin, rall, metab, normb, acc):
    c = lax.axis_index("c")
    s = lax.axis_index("s")
    tid = c * NS + s

    pltpu.sync_copy(nin_hbm.at[pl.ds(tid * CPT * N, CPT * N)], nin)
    pltpu.sync_copy(rall_hbm.at[pl.ds(tid * CPT * RP, CPT * RP)], rall)

    zero = jnp.zeros((LANES,), jnp.float32)

    def z_body(i, carry):
        acc[pl.ds(i * LANES, LANES)] = zero
        return carry
    lax.fori_loop(0, CPT * ROWS // LANES, z_body, 0)

    def c_body(ch, carry):
        pltpu.sync_copy(meta_hbm.at[pl.ds(ch * 2 * CH, 2 * CH)], metab)
        pltpu.sync_copy(norm_hbm.at[pl.ds(ch * CH, CH)], normb)

        def g_body(g, carry2):
            sl = pl.ds(g * LANES, LANES)
            ps_v = metab[sl]
            src_v = ps_v & 16383
            ret_v = (ps_v >> 14) & 15
            sidx_v = metab[pl.ds(CH + g * LANES, LANES)]
            norm_v = normb[sl]
            for col in range(CPT):
                nin_v = plsc.load_gather(nin, [src_v + col * N])
                ral_v = plsc.load_gather(rall, [ret_v + col * RP])
                val = nin_v * ral_v * norm_v
                plsc.addupdate_scatter(acc, [sidx_v + col * ROWS], val)
            return carry2
        lax.fori_loop(0, CH // LANES, g_body, 0)
        return carry
    lax.fori_loop(0, NCHUNK, c_body, 0)

    pltpu.sync_copy(acc, out_hbm.at[pl.ds(tid * CPT * ROWS, CPT * ROWS)])


@functools.partial(
    pl.kernel,
    out_type=jax.ShapeDtypeStruct((D * ROWS,), jnp.float32),
    mesh=plsc.VectorSubcoreMesh(core_axis_name="c", subcore_axis_name="s"),
    compiler_params=pltpu.CompilerParams(needs_layout_passes=False),
    scratch_types=[
        pltpu.VMEM((CPT * N,), jnp.float32),
        pltpu.VMEM((CPT * RP,), jnp.float32),
        pltpu.VMEM((2 * CH,), jnp.int32),
        pltpu.VMEM((CH,), jnp.float32),
        pltpu.VMEM((CPT * ROWS,), jnp.float32),
    ],
)
def _sc_edge_kernel(nin_hbm, rall_hbm, meta_hbm, norm_hbm, out_hbm,
                    nin, rall, metab, normb, acc):
    _sc_body(nin_hbm, rall_hbm, meta_hbm, norm_hbm, out_hbm,
             nin, rall, metab, normb, acc)


def _final_body(x_ref, ai_ref, ao_ref, wit_ref, wot_ref, wst_ref, loop_ref,
                rf_ref, wrt_ref, o_ref, r_ref):
    i = pl.program_id(0)
    xs = x_ref[...] * loop_ref[...]
    acc = jnp.dot(xs, wst_ref[...], preferred_element_type=jnp.float32)
    acc += jnp.dot(ai_ref[...], wit_ref[...], preferred_element_type=jnp.float32)
    acc += jnp.dot(ao_ref[...], wot_ref[...], preferred_element_type=jnp.float32)
    o_ref[...] = jnp.tanh(acc * (1.0 / 3.0))

    @pl.when(i == 0)
    def _():
        r_ref[...] = jnp.dot(rf_ref[...], wrt_ref[...],
                             preferred_element_type=jnp.float32)


def _final(n_in_feats, a_t, wi_t, wo_t, ws_t, loop_rel, r_feats, wr_t):
    bn = 1000
    nb = N // bn
    return pl.pallas_call(
        _final_body,
        grid=(nb,),
        in_specs=[
            pl.BlockSpec((bn, D), lambda i: (i, 0)),
            pl.BlockSpec((bn, D), lambda i: (i, 0)),
            pl.BlockSpec((bn, D), lambda i: (nb + i, 0)),
            pl.BlockSpec((D, D), lambda i: (0, 0)),
            pl.BlockSpec((D, D), lambda i: (0, 0)),
            pl.BlockSpec((D, D), lambda i: (0, 0)),
            pl.BlockSpec((1, D), lambda i: (0, 0)),
            pl.BlockSpec((R, D), lambda i: (0, 0)),
            pl.BlockSpec((D, D), lambda i: (0, 0)),
        ],
        out_specs=[
            pl.BlockSpec((bn, D), lambda i: (i, 0)),
            pl.BlockSpec((R, D), lambda i: (0, 0)),
        ],
        out_shape=[
            jax.ShapeDtypeStruct((N, D), jnp.float32),
            jax.ShapeDtypeStruct((R, D), jnp.float32),
        ],
    )(n_in_feats, a_t, a_t, wi_t, wo_t, ws_t, loop_rel, r_feats, wr_t)


def kernel(n_in_feats, r_feats, edge_index, etype, norm, out_edges_mask,
           W_O, W_I, W_S, W_R, loop_rel):
    r_all = jnp.concatenate([r_feats, loop_rel], axis=0)
    nin_t = n_in_feats.T.reshape(-1)
    rall_t = jnp.pad(r_all.T, ((0, 0), (0, RP - (R + 1)))).reshape(-1)

    src = edge_index[0]
    dst = edge_index[1]
    ps = src | (etype << 14)
    sidx = out_edges_mask * N + dst
    meta = jnp.stack([ps.reshape(NCHUNK, CH), sidx.reshape(NCHUNK, CH)],
                     axis=1).reshape(-1)
    norm_e = norm[:, 0]

    flat = _sc_edge_kernel(nin_t, rall_t, meta, norm_e)
    a_t = flat.reshape(D, ROWS).T

    n_out, r_out = _final(n_in_feats, a_t, W_I.T, W_O.T, W_S.T, loop_rel,
                          r_feats, W_R.T)
    return n_out, r_out


# speedup vs baseline: 2.9889x; 1.0180x over previous
import functools

import jax
import jax.numpy as jnp
from jax import lax
from jax.experimental import pallas as pl
from jax.experimental.pallas import tpu as pltpu
from jax.experimental.pallas import tpu_sc as plsc

N = 10000
E = 320000
D = 128
R = 8

NC = 2
NS = 16
NW = NC * NS
LANES = 16

CPT = D // NW
ROWS = 2 * N
CH = 3200
NCHUNK = E // CH
RP = 16


def _sc_body(nin_hbm, rall_hbm, meta_hbm, norm_hbm, out_hbm,
             nin, rall, metab, normb, acc):
    c = lax.axis_index("c")
    s = lax.axis_index("s")
    tid = c * NS + s

    pltpu.sync_copy(nin_hbm.at[pl.ds(tid * CPT * N, CPT * N)], nin)
    pltpu.sync_copy(rall_hbm.at[pl.ds(tid * CPT * RP, CPT * RP)], rall)

    zero = jnp.zeros((LANES,), jnp.float32)

    def z_body(i, carry):
        acc[pl.ds(i * LANES, LANES)] = zero
        return carry
    lax.fori_loop(0, CPT * ROWS // LANES, z_body, 0)

    def c_body(ch, carry):
        pltpu.sync_copy(meta_hbm.at[pl.ds(ch * 2 * CH, 2 * CH)], metab)
        pltpu.sync_copy(norm_hbm.at[pl.ds(ch * CH, CH)], normb)

        def g_body(g, carry2):
            for u in range(2):
                off = (g * 2 + u) * LANES
                sl = pl.ds(off, LANES)
                ps_v = metab[sl]
                src_v = ps_v & 16383
                ret_v = (ps_v >> 14) & 15
                sidx_v = metab[pl.ds(CH + off, LANES)]
                norm_v = normb[sl]
                for col in range(CPT):
                    nin_v = plsc.load_gather(nin, [src_v + col * N])
                    ral_v = plsc.load_gather(rall, [ret_v + col * RP])
                    val = nin_v * ral_v * norm_v
                    plsc.addupdate_scatter(acc, [sidx_v + col * ROWS], val)
            return carry2
        lax.fori_loop(0, CH // LANES // 2, g_body, 0)
        return carry
    lax.fori_loop(0, NCHUNK, c_body, 0)

    pltpu.sync_copy(acc, out_hbm.at[pl.ds(tid * CPT * ROWS, CPT * ROWS)])


@functools.partial(
    pl.kernel,
    out_type=jax.ShapeDtypeStruct((D * ROWS,), jnp.float32),
    mesh=plsc.VectorSubcoreMesh(core_axis_name="c", subcore_axis_name="s"),
    compiler_params=pltpu.CompilerParams(needs_layout_passes=False),
    scratch_types=[
        pltpu.VMEM((CPT * N,), jnp.float32),
        pltpu.VMEM((CPT * RP,), jnp.float32),
        pltpu.VMEM((2 * CH,), jnp.int32),
        pltpu.VMEM((CH,), jnp.float32),
        pltpu.VMEM((CPT * ROWS,), jnp.float32),
    ],
)
def _sc_edge_kernel(nin_hbm, rall_hbm, meta_hbm, norm_hbm, out_hbm,
                    nin, rall, metab, normb, acc):
    _sc_body(nin_hbm, rall_hbm, meta_hbm, norm_hbm, out_hbm,
             nin, rall, metab, normb, acc)


def _final_body(x_ref, ai_ref, ao_ref, wit_ref, wot_ref, wst_ref, loop_ref,
                rf_ref, wrt_ref, o_ref, r_ref):
    i = pl.program_id(0)
    xs = x_ref[...] * loop_ref[...]
    acc = jnp.dot(xs, wst_ref[...], preferred_element_type=jnp.float32)
    acc += jnp.dot(ai_ref[...], wit_ref[...], preferred_element_type=jnp.float32)
    acc += jnp.dot(ao_ref[...], wot_ref[...], preferred_element_type=jnp.float32)
    o_ref[...] = jnp.tanh(acc * (1.0 / 3.0))

    @pl.when(i == 0)
    def _():
        r_ref[...] = jnp.dot(rf_ref[...], wrt_ref[...],
                             preferred_element_type=jnp.float32)


def _final(n_in_feats, a_t, wi_t, wo_t, ws_t, loop_rel, r_feats, wr_t):
    bn = 1000
    nb = N // bn
    return pl.pallas_call(
        _final_body,
        grid=(nb,),
        in_specs=[
            pl.BlockSpec((bn, D), lambda i: (i, 0)),
            pl.BlockSpec((bn, D), lambda i: (i, 0)),
            pl.BlockSpec((bn, D), lambda i: (nb + i, 0)),
            pl.BlockSpec((D, D), lambda i: (0, 0)),
            pl.BlockSpec((D, D), lambda i: (0, 0)),
            pl.BlockSpec((D, D), lambda i: (0, 0)),
            pl.BlockSpec((1, D), lambda i: (0, 0)),
            pl.BlockSpec((R, D), lambda i: (0, 0)),
            pl.BlockSpec((D, D), lambda i: (0, 0)),
        ],
        out_specs=[
            pl.BlockSpec((bn, D), lambda i: (i, 0)),
            pl.BlockSpec((R, D), lambda i: (0, 0)),
        ],
        out_shape=[
            jax.ShapeDtypeStruct((N, D), jnp.float32),
            jax.ShapeDtypeStruct((R, D), jnp.float32),
        ],
    )(n_in_feats, a_t, a_t, wi_t, wo_t, ws_t, loop_rel, r_feats, wr_t)


def kernel(n_in_feats, r_feats, edge_index, etype, norm, out_edges_mask,
           W_O, W_I, W_S, W_R, loop_rel):
    r_all = jnp.concatenate([r_feats, loop_rel], axis=0)
    nin_t = n_in_feats.T.reshape(-1)
    rall_t = jnp.pad(r_all.T, ((0, 0), (0, RP - (R + 1)))).reshape(-1)

    src = edge_index[0]
    dst = edge_index[1]
    ps = src | (etype << 14)
    sidx = out_edges_mask * N + dst
    meta = jnp.stack([ps.reshape(NCHUNK, CH), sidx.reshape(NCHUNK, CH)],
                     axis=1).reshape(-1)
    norm_e = norm[:, 0]

    flat = _sc_edge_kernel(nin_t, rall_t, meta, norm_e)
    a_t = flat.reshape(D, ROWS).T

    n_out, r_out = _final(n_in_feats, a_t, W_I.T, W_O.T, W_S.T, loop_rel,
                          r_feats, W_R.T)
    return n_out, r_out
